# Initial kernel scaffold; baseline (speedup 1.0000x reference)
import jax
import jax.numpy as jnp
from jax.experimental import pallas as pl


def kernel(examples, labels, embeddings, label_embeddings):
    raise NotImplementedError("write your pallas kernel here")


# speedup vs baseline: 2.0219x; 2.0219x over previous
import functools

import jax
import jax.numpy as jnp
from jax import lax
from jax.experimental import pallas as pl
from jax.experimental.pallas import tpu as pltpu
from jax.experimental.pallas import tpu_sc as plsc

S = 1024
N = 50
D = 128
SEQ = 3 * N - 1
NC = 2
NS = 16
NW = NC * NS

EX_TOTAL = S * 2 * N
LB_TOTAL = S * (N - 1)
EX_K = 128
LB_K = 112
EX_ROWS = EX_TOTAL // EX_K
LB_ROWS = LB_TOTAL // LB_K
EX_PER_W = EX_ROWS // NW
LB_PER_W = LB_ROWS // NW


def _sc_gather_kernel():
    mesh = plsc.VectorSubcoreMesh(core_axis_name="c", subcore_axis_name="s")

    @functools.partial(
        pl.kernel,
        mesh=mesh,
        out_type=jax.ShapeDtypeStruct((S * SEQ, D), jnp.float32),
        scratch_types=[
            pltpu.VMEM((EX_K,), jnp.int32),
            pltpu.VMEM((EX_K,), jnp.int32),
            pltpu.VMEM((EX_K, D), jnp.float32),
            pltpu.VMEM((LB_K,), jnp.int32),
            pltpu.VMEM((LB_K,), jnp.int32),
            pltpu.VMEM((LB_K, D), jnp.float32),
            pltpu.SemaphoreType.DMA,
            pltpu.SemaphoreType.DMA,
        ],
    )
    def k(emb, lemb, ex_src, ex_dst, lb_src, lb_dst, out,
          sidx_e, didx_e, rows_e, sidx_l, didx_l, rows_l, sem_g, sem_s):
        wid = lax.axis_index("s") * NC + lax.axis_index("c")

        ex_base = wid * EX_PER_W

        def ex_it(j, carry):
            r = ex_base + j
            pltpu.sync_copy(ex_src.at[r], sidx_e)
            pltpu.sync_copy(ex_dst.at[r], didx_e)
            pltpu.async_copy(emb.at[sidx_e], rows_e, sem_g).wait()
            pltpu.async_copy(rows_e, out.at[didx_e], sem_s).wait()
            return carry

        lax.fori_loop(0, EX_PER_W, ex_it, 0, unroll=False)

        lb_base = wid * LB_PER_W

        def lb_it(j, carry):
            r = lb_base + j
            pltpu.sync_copy(lb_src.at[r], sidx_l)
            pltpu.sync_copy(lb_dst.at[r], didx_l)
            pltpu.async_copy(lemb.at[sidx_l], rows_l, sem_g).wait()
            pltpu.async_copy(rows_l, out.at[didx_l], sem_s).wait()
            return carry

        lax.fori_loop(0, LB_PER_W, lb_it, 0, unroll=False)

    return k


_KERNEL = _sc_gather_kernel()


def kernel(examples, labels, embeddings, label_embeddings):
    ex_src = examples.reshape(EX_ROWS, EX_K)
    lb_src = labels[:, : N - 1].reshape(LB_ROWS, LB_K)

    b = jnp.arange(S, dtype=jnp.int32)[:, None]
    j = jnp.arange(2 * N, dtype=jnp.int32)[None, :]
    ex_dst = (b * SEQ + 3 * (j // 2) + (j % 2)).reshape(EX_ROWS, EX_K)
    jl = jnp.arange(N - 1, dtype=jnp.int32)[None, :]
    lb_dst = (b * SEQ + 3 * jl + 2).reshape(LB_ROWS, LB_K)

    out = _KERNEL(embeddings, label_embeddings, ex_src, ex_dst, lb_src, lb_dst)
    return out.reshape(S, SEQ, D)


# speedup vs baseline: 2.5174x; 1.2451x over previous
import functools

import jax
import jax.numpy as jnp
from jax import lax
from jax.experimental import pallas as pl
from jax.experimental.pallas import tpu as pltpu
from jax.experimental.pallas import tpu_sc as plsc

S = 1024
N = 50
D = 128
SEQ = 3 * N - 1
NC = 2
NS = 16
NW = NC * NS

EX_TOTAL = S * 2 * N
LB_TOTAL = S * (N - 1)
EX_K = 128
LB_K = 112
EX_ROWS = EX_TOTAL // EX_K
LB_ROWS = LB_TOTAL // LB_K
EX_PER_W = EX_ROWS // NW
LB_PER_W = LB_ROWS // NW
EX_NBUF = 5
LB_NBUF = 2


def _sc_gather_kernel():
    mesh = plsc.VectorSubcoreMesh(core_axis_name="c", subcore_axis_name="s")

    @functools.partial(
        pl.kernel,
        mesh=mesh,
        out_type=jax.ShapeDtypeStruct((S * SEQ, D), jnp.float32),
        scratch_types=(
            [pltpu.VMEM((EX_K, D), jnp.float32) for _ in range(EX_NBUF)]
            + [
                pltpu.VMEM((EX_PER_W, EX_K), jnp.int32),
                pltpu.VMEM((EX_PER_W, EX_K), jnp.int32),
                pltpu.VMEM((LB_PER_W, LB_K), jnp.int32),
                pltpu.VMEM((LB_PER_W, LB_K), jnp.int32),
            ]
            + [pltpu.SemaphoreType.DMA for _ in range(2 * EX_NBUF)]
        ),
    )
    def k(emb, lemb, ex_src, ex_dst, lb_src, lb_dst, out, *scratch):
        bufs = scratch[:EX_NBUF]
        sidx_e, didx_e, sidx_l, didx_l = scratch[EX_NBUF:EX_NBUF + 4]
        sem_g = scratch[EX_NBUF + 4:EX_NBUF + 4 + EX_NBUF]
        sem_s = scratch[EX_NBUF + 4 + EX_NBUF:]

        wid = lax.axis_index("s") * NC + lax.axis_index("c")

        pltpu.sync_copy(ex_src.at[wid], sidx_e)
        pltpu.sync_copy(ex_dst.at[wid], didx_e)
        pltpu.sync_copy(lb_src.at[wid], sidx_l)
        pltpu.sync_copy(lb_dst.at[wid], didx_l)

        def run_stream(tbl, sidx, didx, n, K, nbuf):
            laps = n // nbuf

            def buf(b):
                if K == EX_K:
                    return bufs[b]
                return bufs[b].at[pl.ds(0, K)]

            def start_gather(j, b):
                pltpu.async_copy(tbl.at[sidx.at[j]], buf(b), sem_g[b])

            def wait_gather(j, b):
                pltpu.make_async_copy(tbl.at[sidx.at[j]], buf(b),
                                      sem_g[b]).wait()

            def start_scatter(j, b):
                pltpu.async_copy(buf(b), out.at[didx.at[j]], sem_s[b])

            def wait_scatter(j, b):
                pltpu.make_async_copy(buf(b), out.at[didx.at[j]],
                                      sem_s[b]).wait()

            for b in range(nbuf):
                start_gather(b, b)

            def lap(t, carry):
                for b in range(nbuf):
                    j = t * nbuf + b
                    wait_gather(j, b)
                    start_scatter(j, b)
                    wait_scatter(j, b)
                    start_gather(j + nbuf, b)
                return carry

            if laps > 1:
                lax.fori_loop(0, laps - 1, lap, 0, unroll=False)
            for b in range(nbuf):
                j = (laps - 1) * nbuf + b
                wait_gather(j, b)
                start_scatter(j, b)
            for b in range(nbuf):
                j = (laps - 1) * nbuf + b
                wait_scatter(j, b)

        run_stream(emb, sidx_e, didx_e, EX_PER_W, EX_K, EX_NBUF)
        run_stream(lemb, sidx_l, didx_l, LB_PER_W, LB_K, LB_NBUF)

    return k


_KERNEL = _sc_gather_kernel()


def kernel(examples, labels, embeddings, label_embeddings):
    ex_src = examples.reshape(NW, EX_PER_W, EX_K)
    lb_src = labels[:, : N - 1].reshape(NW, LB_PER_W, LB_K)

    b = jnp.arange(S, dtype=jnp.int32)[:, None]
    j = jnp.arange(2 * N, dtype=jnp.int32)[None, :]
    ex_dst = (b * SEQ + 3 * (j // 2) + (j % 2)).reshape(NW, EX_PER_W, EX_K)
    jl = jnp.arange(N - 1, dtype=jnp.int32)[None, :]
    lb_dst = (b * SEQ + 3 * jl + 2).reshape(NW, LB_PER_W, LB_K)

    out = _KERNEL(embeddings, label_embeddings, ex_src, ex_dst, lb_src, lb_dst)
    return out.reshape(S, SEQ, D)


# speedup vs baseline: 3.6340x; 1.4435x over previous
import functools

import jax
import jax.numpy as jnp
from jax import lax
from jax.experimental import pallas as pl
from jax.experimental.pallas import tpu as pltpu
from jax.experimental.pallas import tpu_sc as plsc

S = 1024
N = 50
D = 128
SEQ = 3 * N - 1
PSEQ = 152
NC = 2
NS = 16
NW = NC * NS

EX_TOTAL = S * 2 * N
LB_TOTAL = S * (N - 1)
EX_K = 128
LB_K = 112
EX_ROWS = EX_TOTAL // EX_K
LB_ROWS = LB_TOTAL // LB_K
EX_PER_W = EX_ROWS // NW
LB_PER_W = LB_ROWS // NW
EX_NBUF = 5
LB_NBUF = 2


def _sc_gather_kernel():
    mesh = plsc.VectorSubcoreMesh(core_axis_name="c", subcore_axis_name="s")

    @functools.partial(
        pl.kernel,
        mesh=mesh,
        out_type=jax.ShapeDtypeStruct((S * PSEQ, D), jnp.float32),
        scratch_types=(
            [pltpu.VMEM((EX_K, D), jnp.float32) for _ in range(EX_NBUF)]
            + [
                pltpu.VMEM((EX_PER_W, EX_K), jnp.int32),
                pltpu.VMEM((EX_PER_W, EX_K), jnp.int32),
                pltpu.VMEM((LB_PER_W, LB_K), jnp.int32),
                pltpu.VMEM((LB_PER_W, LB_K), jnp.int32),
            ]
            + [pltpu.SemaphoreType.DMA for _ in range(2 * EX_NBUF)]
        ),
    )
    def k(emb, lemb, ex_src, ex_dst, lb_src, lb_dst, out, *scratch):
        bufs = scratch[:EX_NBUF]
        sidx_e, didx_e, sidx_l, didx_l = scratch[EX_NBUF:EX_NBUF + 4]
        sem_g = scratch[EX_NBUF + 4:EX_NBUF + 4 + EX_NBUF]
        sem_s = scratch[EX_NBUF + 4 + EX_NBUF:]

        wid = lax.axis_index("s") * NC + lax.axis_index("c")

        pltpu.sync_copy(ex_src.at[wid], sidx_e)
        pltpu.sync_copy(ex_dst.at[wid], didx_e)
        pltpu.sync_copy(lb_src.at[wid], sidx_l)
        pltpu.sync_copy(lb_dst.at[wid], didx_l)

        def run_stream(tbl, sidx, didx, n, K, nbuf):
            laps = n // nbuf

            def buf(b):
                if K == EX_K:
                    return bufs[b]
                return bufs[b].at[pl.ds(0, K)]

            def start_gather(j, b):
                pltpu.async_copy(tbl.at[sidx.at[j]], buf(b), sem_g[b])

            def wait_gather(j, b):
                pltpu.make_async_copy(tbl.at[sidx.at[j]], buf(b),
                                      sem_g[b]).wait()

            def start_scatter(j, b):
                pltpu.async_copy(buf(b), out.at[didx.at[j]], sem_s[b])

            def wait_scatter(j, b):
                pltpu.make_async_copy(buf(b), out.at[didx.at[j]],
                                      sem_s[b]).wait()

            for b in range(nbuf):
                start_gather(b, b)

            def lap(t, carry):
                for b in range(nbuf):
                    j = t * nbuf + b
                    wait_gather(j, b)
                    start_scatter(j, b)
                    wait_scatter(j, b)
                    start_gather(j + nbuf, b)
                return carry

            if laps > 1:
                lax.fori_loop(0, laps - 1, lap, 0, unroll=False)
            for b in range(nbuf):
                j = (laps - 1) * nbuf + b
                wait_gather(j, b)
                start_scatter(j, b)
            for b in range(nbuf):
                j = (laps - 1) * nbuf + b
                wait_scatter(j, b)

        run_stream(emb, sidx_e, didx_e, EX_PER_W, EX_K, EX_NBUF)
        run_stream(lemb, sidx_l, didx_l, LB_PER_W, LB_K, LB_NBUF)

    return k


_KERNEL = _sc_gather_kernel()


def kernel(examples, labels, embeddings, label_embeddings):
    ex_src = examples.reshape(NW, EX_PER_W, EX_K)
    lb_src = labels[:, : N - 1].reshape(NW, LB_PER_W, LB_K)

    b = jnp.arange(S, dtype=jnp.int32)[:, None]
    j = jnp.arange(2 * N, dtype=jnp.int32)[None, :]
    ex_dst = (b * PSEQ + 3 * (j // 2) + (j % 2)).reshape(NW, EX_PER_W, EX_K)
    jl = jnp.arange(N - 1, dtype=jnp.int32)[None, :]
    lb_dst = (b * PSEQ + 3 * jl + 2).reshape(NW, LB_PER_W, LB_K)

    out = _KERNEL(embeddings, label_embeddings, ex_src, ex_dst, lb_src, lb_dst)
    return out.reshape(S, PSEQ, D)[:, :SEQ]


# speedup vs baseline: 6.0937x; 1.6769x over previous
import functools

import jax
import jax.numpy as jnp
from jax import lax
from jax.experimental import pallas as pl
from jax.experimental.pallas import tpu as pltpu
from jax.experimental.pallas import tpu_sc as plsc

S = 1024
N = 50
D = 128
SEQ = 3 * N - 1
NC = 2
NS = 16
NW = NC * NS

CK = 128
CPC = S // CK
EX_CHUNKS = 2 * N * CPC
LB_REAL = (N - 1) * CPC
EX_PER_W = EX_CHUNKS // NW
LB_PER_W = 13
LB_PAD = NW * LB_PER_W - LB_REAL
EX_NBUF = 5


def _sc_gather_kernel():
    mesh = plsc.VectorSubcoreMesh(core_axis_name="c", subcore_axis_name="s")

    @functools.partial(
        pl.kernel,
        mesh=mesh,
        out_type=jax.ShapeDtypeStruct((SEQ * S, D), jnp.float32),
        scratch_types=(
            [pltpu.VMEM((CK, D), jnp.float32) for _ in range(EX_NBUF)]
            + [
                pltpu.VMEM((EX_PER_W, CK), jnp.int32),
                pltpu.VMEM((LB_PER_W, CK), jnp.int32),
            ]
            + [pltpu.SemaphoreType.DMA for _ in range(2 * EX_NBUF)]
        ),
    )
    def k(emb, lemb, ex_src, lb_src, out, *scratch):
        bufs = scratch[:EX_NBUF]
        sidx_e, sidx_l = scratch[EX_NBUF:EX_NBUF + 2]
        sem_g = scratch[EX_NBUF + 2:EX_NBUF + 2 + EX_NBUF]
        sem_s = scratch[EX_NBUF + 2 + EX_NBUF:]

        wid = lax.axis_index("s") * NC + lax.axis_index("c")

        pltpu.sync_copy(ex_src.at[wid], sidx_e)
        pltpu.sync_copy(lb_src.at[wid], sidx_l)

        def ex_base(j):
            g = wid * EX_PER_W + j
            col = g >> 3
            seg = g & 7
            t = 3 * (col >> 1) + (col & 1)
            return t * S + seg * CK

        def lb_base(i):
            g = wid * LB_PER_W + i
            g = jnp.where(g < LB_REAL, g, g - LB_REAL)
            col = g >> 3
            seg = g & 7
            t = 3 * col + 2
            return t * S + seg * CK

        def start_gather(tbl, sidx, j, b):
            pltpu.async_copy(tbl.at[sidx.at[j]], bufs[b], sem_g[b])

        def wait_gather(tbl, sidx, j, b):
            pltpu.make_async_copy(tbl.at[sidx.at[j]], bufs[b],
                                  sem_g[b]).wait()

        def start_scatter(base, b):
            pltpu.async_copy(bufs[b], out.at[pl.ds(base, CK)], sem_s[b])

        def wait_scatter(base, b):
            pltpu.make_async_copy(bufs[b], out.at[pl.ds(base, CK)],
                                  sem_s[b]).wait()

        for b in range(EX_NBUF):
            start_gather(emb, sidx_e, b, b)

        def lap(t, carry):
            for b in range(EX_NBUF):
                j = t * EX_NBUF + b
                wait_gather(emb, sidx_e, j, b)
                start_scatter(ex_base(j), b)
                wait_scatter(ex_base(j), b)
                start_gather(emb, sidx_e, j + EX_NBUF, b)
            return carry

        laps = EX_PER_W // EX_NBUF
        lax.fori_loop(0, laps - 1, lap, 0, unroll=False)
        for b in range(EX_NBUF):
            j = (laps - 1) * EX_NBUF + b
            wait_gather(emb, sidx_e, j, b)
            start_scatter(ex_base(j), b)
        for b in range(EX_NBUF):
            wait_scatter(ex_base((laps - 1) * EX_NBUF + b), b)

        for i in range(min(EX_NBUF, LB_PER_W)):
            start_gather(lemb, sidx_l, i, i)
        for i in range(LB_PER_W):
            b = i % EX_NBUF
            wait_gather(lemb, sidx_l, i, b)
            start_scatter(lb_base(i), b)
            nxt = i + EX_NBUF
            if nxt < LB_PER_W:
                wait_scatter(lb_base(i), b)
                start_gather(lemb, sidx_l, nxt, b)
        for i in range(LB_PER_W - EX_NBUF, LB_PER_W):
            wait_scatter(lb_base(i), i % EX_NBUF)

    return k


_KERNEL = _sc_gather_kernel()


def kernel(examples, labels, embeddings, label_embeddings):
    ex_src = examples.T.reshape(NW, EX_PER_W, CK)
    lb_flat = labels[:, : N - 1].T.reshape(LB_REAL, CK)
    lb_src = jnp.concatenate([lb_flat, lb_flat[:LB_PAD]]
                             ).reshape(NW, LB_PER_W, CK)

    out = _KERNEL(embeddings, label_embeddings, ex_src, lb_src)
    return jnp.swapaxes(out.reshape(SEQ, S, D), 0, 1)
